# baseline (speedup 1.0000x reference)
import functools

import jax
import jax.numpy as jnp
from jax import lax
from jax.experimental import pallas as pl
from jax.experimental.pallas import tpu as pltpu
from jax.experimental.pallas import tpu_sc as plsc

B, L, F, D = 4096, 50, 5, 64
N = B * L
V = 100000

NC, NS = 2, 16
NW = NC * NS
RPW = N // NW
CH = 640
NB = RPW // CH


def _emb(idxT, tcat):
    mesh = plsc.VectorSubcoreMesh(core_axis_name="c", subcore_axis_name="s")

    @functools.partial(
        pl.kernel,
        out_type=jax.ShapeDtypeStruct((N, F, D), jnp.float32),
        mesh=mesh,
        scratch_types=[
            pltpu.VMEM((F * RPW,), jnp.int32),
            pltpu.VMEM((CH, 1, D), jnp.float32),
            pltpu.VMEM((CH, 1, D), jnp.float32),
            pltpu.SemaphoreType.DMA,
            pltpu.SemaphoreType.DMA,
            pltpu.SemaphoreType.DMA,
            pltpu.SemaphoreType.DMA,
        ],
        compiler_params=pltpu.CompilerParams(use_tc_tiling_on_sc=False),
    )
    def body(idx_hbm, tab, out_hbm, idx_all, rows0, rows1, gs0, gs1, ss0, ss1):
        bufs, gsem, ssem = [rows0, rows1], [gs0, gs1], [ss0, ss1]
        wid = lax.axis_index("s") * NC + lax.axis_index("c")
        wbase = pl.multiple_of(wid * RPW, 8)

        for f in range(F):
            pltpu.sync_copy(idx_hbm.at[pl.ds(f * N + wbase, RPW)],
                            idx_all.at[pl.ds(f * RPW, RPW)])

        T = F * NB
        gath, scat = [None, None], [None, None]

        def start_gather(t):
            b = t % 2
            idx = idx_all.at[pl.ds(t * CH, CH)]
            gath[b] = pltpu.async_copy(tab.at[idx], bufs[b].at[:, 0], gsem[b])

        def start_scatter(t):
            f, i, b = t // NB, t % NB, t % 2
            n0 = pl.multiple_of(wbase + i * CH, 8)
            scat[b] = pltpu.async_copy(
                bufs[b], out_hbm.at[pl.ds(n0, CH), pl.ds(f, 1)], ssem[b])

        start_gather(0)
        for t in range(T):
            b, nb = t % 2, (t + 1) % 2
            if t + 1 < T:
                if scat[nb] is not None:
                    scat[nb].wait()
                start_gather(t + 1)
            gath[b].wait()
            start_scatter(t)
        scat[0].wait()
        scat[1].wait()

    return body(idxT, tcat)


def kernel(input, T0, T1, T2, T3, T4):
    offs = (jnp.arange(F, dtype=jnp.int32) * V)[None, None, :]
    idxT = (input + offs).transpose(2, 0, 1).reshape(-1)
    tcat = jnp.concatenate([T0, T1, T2, T3, T4], axis=0)
    out = _emb(idxT, tcat)
    return out.reshape(B, L, F, D)
